# Initial kernel scaffold; baseline (speedup 1.0000x reference)
import jax
import jax.numpy as jnp
from jax.experimental import pallas as pl


def kernel(x, edge_index, W_base, b_base, W1, b1, W2, b2):
    raise NotImplementedError("write your pallas kernel here")


# speedup vs baseline: 1.6766x; 1.6766x over previous
import functools

import jax
import jax.numpy as jnp
from jax import lax
from jax.experimental import pallas as pl
from jax.experimental.pallas import tpu as pltpu
from jax.experimental.pallas import tpu_sc as plsc

NC = 2
NS = 16
LANES = 16
CH = 128


def _mesh():
    return plsc.VectorSubcoreMesh(
        core_axis_name="c", subcore_axis_name="s", num_cores=NC, num_subcores=NS
    )


def _p1_linear(x, W, b):
    N, C = x.shape
    BLK = 2000

    def body(x_ref, w_ref, b_ref, h_ref):
        h_ref[...] = jnp.dot(x_ref[...], w_ref[...],
                             preferred_element_type=jnp.float32) + b_ref[...]

    return pl.pallas_call(
        body,
        grid=(N // BLK,),
        in_specs=[
            pl.BlockSpec((BLK, C), lambda i: (i, 0)),
            pl.BlockSpec((C, C), lambda i: (0, 0)),
            pl.BlockSpec((1, C), lambda i: (0, 0)),
        ],
        out_specs=pl.BlockSpec((BLK, C), lambda i: (i, 0)),
        out_shape=jax.ShapeDtypeStruct((N, C), jnp.float32),
    )(x, W, b)


def _padd(p0, p1, n_out):
    rows, C = p0.shape
    BLK = next(b for b in (2528, 2000, 1264, 1000, 632, 200, 8)
               if n_out % b == 0)

    def body(a_ref, b_ref, o_ref):
        o_ref[...] = a_ref[...] + b_ref[...]

    return pl.pallas_call(
        body,
        grid=(n_out // BLK,),
        in_specs=[
            pl.BlockSpec((BLK, C), lambda i: (i, 0)),
            pl.BlockSpec((BLK, C), lambda i: (i, 0)),
        ],
        out_specs=pl.BlockSpec((BLK, C), lambda i: (i, 0)),
        out_shape=jax.ShapeDtypeStruct((n_out, C), jnp.float32),
    )(p0, p1)


def _p2_scatter(h, srcp, dstp, zrows, n_chunks, n_acc, rows_t):
    N, C = h.shape

    @functools.partial(
        pl.kernel,
        out_type=[
            jax.ShapeDtypeStruct((n_acc, C), jnp.float32),
            jax.ShapeDtypeStruct((n_acc, C), jnp.float32),
        ],
        mesh=_mesh(),
        scratch_types=[
            pltpu.VMEM((CH,), jnp.int32),
            pltpu.VMEM((CH,), jnp.int32),
            pltpu.VMEM((CH, C), jnp.float32),
            pltpu.VMEM_SHARED((n_acc, C), jnp.float32),
            pltpu.SemaphoreType.DMA,
        ],
    )
    def k(h_hbm, src_hbm, dst_hbm, z_hbm, p0_hbm, p1_hbm,
          sidx, didx, rows, acc, sem):
        c = lax.axis_index("c")
        s = lax.axis_index("s")
        pltpu.sync_copy(z_hbm, acc.at[pl.ds(s * rows_t, rows_t)])
        plsc.subcore_barrier()

        def chunk(i, _):
            off = ((c * NS + s) * n_chunks + i) * CH
            pltpu.sync_copy(src_hbm.at[pl.ds(off, CH)], sidx)
            pltpu.async_copy(h_hbm.at[sidx], rows, sem).wait()
            pltpu.sync_copy(dst_hbm.at[pl.ds(off, CH)], didx)
            pltpu.sync_copy(rows, acc.at[didx], add=True)
            return 0

        lax.fori_loop(0, n_chunks, chunk, 0)
        plsc.subcore_barrier()

        @pl.when(c == 0)
        def _():
            pltpu.sync_copy(acc.at[pl.ds(s * rows_t, rows_t)],
                            p0_hbm.at[pl.ds(s * rows_t, rows_t)])

        @pl.when(c == 1)
        def _():
            pltpu.sync_copy(acc.at[pl.ds(s * rows_t, rows_t)],
                            p1_hbm.at[pl.ds(s * rows_t, rows_t)])

    return k(h, srcp, dstp, zrows)


def _p3_gather(emb, srcp, dstp, n_chunks, e_pad):
    N, C = emb.shape

    @functools.partial(
        pl.kernel,
        out_type=[
            jax.ShapeDtypeStruct((e_pad, C), jnp.float32),
            jax.ShapeDtypeStruct((e_pad, C), jnp.float32),
        ],
        mesh=_mesh(),
        scratch_types=[
            pltpu.VMEM((CH,), jnp.int32),
            pltpu.VMEM((CH, C), jnp.float32),
            pltpu.SemaphoreType.DMA,
        ],
    )
    def k(e_hbm, src_hbm, dst_hbm, f1_hbm, f2_hbm, idx, rows, sem):
        c = lax.axis_index("c")
        s = lax.axis_index("s")

        def chunk(i, _):
            off = ((c * NS + s) * n_chunks + i) * CH
            pltpu.sync_copy(src_hbm.at[pl.ds(off, CH)], idx)
            pltpu.async_copy(e_hbm.at[idx], rows, sem).wait()
            pltpu.sync_copy(rows, f1_hbm.at[pl.ds(off, CH)])
            pltpu.sync_copy(dst_hbm.at[pl.ds(off, CH)], idx)
            pltpu.async_copy(e_hbm.at[idx], rows, sem).wait()
            pltpu.sync_copy(rows, f2_hbm.at[pl.ds(off, CH)])
            return 0

        lax.fori_loop(0, n_chunks, chunk, 0)

    return k(emb, srcp, dstp)


def _p4_mlp(f1, f2, Wp, b1r, w2r, b2r):
    e_pad, C = f1.shape
    H = Wp.shape[2]
    BLK = 512

    def body(f1_ref, f2_ref, w_ref, b1_ref, w2_ref, b2_ref, out_ref):
        acc = jnp.dot(f1_ref[...], w_ref[0], preferred_element_type=jnp.float32)
        acc += jnp.dot(f2_ref[...], w_ref[1], preferred_element_type=jnp.float32)
        hb = jnp.maximum(acc + b1_ref[...], 0.0)
        ew = jnp.sum(hb * w2_ref[...], axis=1) + b2_ref[0]
        ew = jnp.maximum(ew, 0.0)
        out_ref[...] = jnp.reshape(ew, (1, 1, BLK))

    return pl.pallas_call(
        body,
        grid=(e_pad // BLK,),
        in_specs=[
            pl.BlockSpec((BLK, C), lambda i: (i, 0)),
            pl.BlockSpec((BLK, C), lambda i: (i, 0)),
            pl.BlockSpec((2, C, H), lambda i: (0, 0, 0)),
            pl.BlockSpec((1, H), lambda i: (0, 0)),
            pl.BlockSpec((1, H), lambda i: (0, 0)),
            pl.BlockSpec(memory_space=pltpu.MemorySpace.SMEM),
        ],
        out_specs=pl.BlockSpec((1, 1, BLK), lambda i: (i, 0, 0)),
        out_shape=jax.ShapeDtypeStruct((e_pad // BLK, 1, BLK), jnp.float32),
    )(f1, f2, Wp, b1r, w2r, b2r)


def _p5_weighted_scatter(h, srcp, dstp, ew, zrows, n_chunks, n_acc, rows_t):
    N, C = h.shape

    @functools.partial(
        pl.kernel,
        out_type=[
            jax.ShapeDtypeStruct((n_acc, C), jnp.float32),
            jax.ShapeDtypeStruct((n_acc, C), jnp.float32),
        ],
        mesh=_mesh(),
        scratch_types=[
            pltpu.VMEM((CH,), jnp.int32),
            pltpu.VMEM((CH,), jnp.int32),
            pltpu.VMEM((CH, C), jnp.float32),
            pltpu.VMEM((CH,), jnp.float32),
            pltpu.VMEM_SHARED((n_acc, C), jnp.float32),
            pltpu.SemaphoreType.DMA,
        ],
        compiler_params=pltpu.CompilerParams(needs_layout_passes=False),
    )
    def k(h_hbm, src_hbm, dst_hbm, ew_hbm, z_hbm, p0_hbm, p1_hbm,
          sidx, didx, rows, ewv, acc, sem):
        c = lax.axis_index("c")
        s = lax.axis_index("s")
        pltpu.sync_copy(z_hbm, acc.at[pl.ds(s * rows_t, rows_t)])
        plsc.subcore_barrier()

        def chunk(i, _):
            off = ((c * NS + s) * n_chunks + i) * CH
            pltpu.sync_copy(src_hbm.at[pl.ds(off, CH)], sidx)
            pltpu.async_copy(h_hbm.at[sidx], rows, sem).wait()
            pltpu.sync_copy(ew_hbm.at[pl.ds(off, CH)], ewv)

            def row(r, _):
                sv = plsc.load_gather(ewv, [jnp.full((LANES,), r, jnp.int32)])
                for j in range(C // LANES):
                    sl = pl.ds(j * LANES, LANES)
                    rows[r, sl] = rows[r, sl] * sv
                return 0

            lax.fori_loop(0, CH, row, 0)
            pltpu.sync_copy(dst_hbm.at[pl.ds(off, CH)], didx)
            pltpu.sync_copy(rows, acc.at[didx], add=True)
            return 0

        lax.fori_loop(0, n_chunks, chunk, 0)
        plsc.subcore_barrier()

        @pl.when(c == 0)
        def _():
            pltpu.sync_copy(acc.at[pl.ds(s * rows_t, rows_t)],
                            p0_hbm.at[pl.ds(s * rows_t, rows_t)])

        @pl.when(c == 1)
        def _():
            pltpu.sync_copy(acc.at[pl.ds(s * rows_t, rows_t)],
                            p1_hbm.at[pl.ds(s * rows_t, rows_t)])

    return k(h, srcp, dstp, ew, zrows)


def kernel(x, edge_index, W_base, b_base, W1, b1, W2, b2):
    N, C = x.shape
    E = edge_index.shape[1]
    H = W1.shape[1]

    n_chunks = -(-E // (NC * NS * CH))
    e_pad = NC * NS * n_chunks * CH
    rows_t = 8 * (-(-(N + 1) // (NS * 8)))
    n_acc = NS * rows_t

    src = edge_index[0].astype(jnp.int32)
    dst = edge_index[1].astype(jnp.int32)
    pad = e_pad - E
    srcp = jnp.concatenate([src, jnp.zeros((pad,), jnp.int32)])
    dstp = jnp.concatenate([dst, jnp.full((pad,), N, jnp.int32)])
    zrows = jnp.zeros((rows_t, C), jnp.float32)

    h = _p1_linear(x, W_base, b_base.reshape(1, C))
    ep0, ep1 = _p2_scatter(h, srcp, dstp, zrows, n_chunks, n_acc, rows_t)
    emb = _padd(ep0, ep1, n_acc)
    f1, f2 = _p3_gather(emb, srcp, dstp, n_chunks, e_pad)

    Wp = jnp.stack([W1[:C], W1[C:]])
    ew_mat = _p4_mlp(f1, f2, Wp, b1.reshape(1, H), W2.reshape(1, H),
                     b2.reshape(1))
    ew_flat = ew_mat.reshape(e_pad)

    lp0, lp1 = _p5_weighted_scatter(h, srcp, dstp, ew_flat, zrows,
                                    n_chunks, n_acc, rows_t)
    return _padd(lp0, lp1, N)
